# baseline (speedup 1.0000x reference)
import numpy as np
import jax
import jax.numpy as jnp
from jax.experimental import pallas as pl

_B = 1024
_F = 26
_D = 64
_BUCKETS = 100000
_OUT = 256
_PI, _PJ = np.triu_indices(_F, k=1)
_P = len(_PI)
_HASH_A = np.uint32([2654435761, 2246822519])
_HASH_B = np.uint32([3266489917, 668265263])
_HASH_C = np.uint32([374761393, 1103515245])

_BB = 128
_PC = 25
_NCHUNK = _P // _PC

_SEL = np.zeros((_F, _F, _P), np.float32)
_SEL[_PI, _PJ, np.arange(_P)] = 1.0

_REP = np.zeros((_PC, _PC * _D), np.float32)
for _p in range(_PC):
    _REP[_p, _p * _D:(_p + 1) * _D] = 1.0


def _prep_body(cb_ref, wm0_ref, wm1_ref, bm_ref, cb0_ref, cb1_ref):
    cb = cb_ref[...]
    cb0_ref[...] = jnp.dot(cb, wm0_ref[...], preferred_element_type=jnp.float32)
    cb1_ref[...] = (
        jnp.dot(cb, wm1_ref[...], preferred_element_type=jnp.float32)
        + bm_ref[...]
    )


def _main_body(emb3_ref, emb2_ref, s2_ref, w1_ref, w2_ref, wa_ref, sel_ref,
               rep_ref, bo_ref, out_ref):
    emb3 = emb3_ref[...]
    wa = wa_ref[...]
    a3 = emb3 * wa
    logits = jnp.zeros((_BB, _P), jnp.float32)
    for i in range(_F):
        gi = jnp.sum(a3[:, i, :][:, None, :] * emb3, axis=-1)
        logits = logits + jnp.dot(gi, sel_ref[i],
                                  preferred_element_type=jnp.float32)
    m = jnp.max(logits, axis=1, keepdims=True)
    e = jnp.exp(logits - m)
    fw = e / jnp.sum(e, axis=1, keepdims=True)

    acc = jnp.dot(emb2_ref[...], w1_ref[...],
                  preferred_element_type=jnp.float32)
    rep = rep_ref[...]
    for c in range(_NCHUNK):
        fwc = fw[:, c * _PC:(c + 1) * _PC]
        fwx = jnp.dot(fwc, rep, preferred_element_type=jnp.float32)
        t = jnp.tanh(s2_ref[:, c * _PC * _D:(c + 1) * _PC * _D]) * fwx
        acc = acc + jnp.dot(t, w2_ref[c * _PC * _D:(c + 1) * _PC * _D, :],
                            preferred_element_type=jnp.float32)
    out_ref[...] = acc + bo_ref[...]


def kernel(feat_inputs, embed_table, codebook, W_merge, b_merge, w_attn,
           W_out, b_out):
    rows = 2000
    cb0, cb1 = pl.pallas_call(
        _prep_body,
        grid=(_BUCKETS // rows,),
        in_specs=[
            pl.BlockSpec((rows, _D), lambda i: (i, 0)),
            pl.BlockSpec((_D, _D), lambda i: (0, 0)),
            pl.BlockSpec((_D, _D), lambda i: (0, 0)),
            pl.BlockSpec((1, _D), lambda i: (0, 0)),
        ],
        out_specs=[
            pl.BlockSpec((rows, _D), lambda i: (i, 0)),
            pl.BlockSpec((rows, _D), lambda i: (i, 0)),
        ],
        out_shape=[
            jax.ShapeDtypeStruct((_BUCKETS, _D), jnp.float32),
            jax.ShapeDtypeStruct((_BUCKETS, _D), jnp.float32),
        ],
    )(codebook, W_merge[:_D], W_merge[_D:], b_merge.reshape(1, _D))

    xi = feat_inputs[:, _PI].astype(jnp.uint32)
    xj = feat_inputs[:, _PJ].astype(jnp.uint32)
    c0 = ((xi * _HASH_A[0] + xj * _HASH_B[0] + _HASH_C[0])
          % jnp.uint32(_BUCKETS)).astype(jnp.int32)
    c1 = ((xi * _HASH_A[1] + xj * _HASH_B[1] + _HASH_C[1])
          % jnp.uint32(_BUCKETS)).astype(jnp.int32)
    s2 = (jnp.take(cb0, c0, axis=0)
          + jnp.take(cb1, c1, axis=0)).reshape(_B, _P * _D)
    emb3 = jnp.take(embed_table, feat_inputs, axis=0)
    emb2 = emb3.reshape(_B, _F * _D)

    out = pl.pallas_call(
        _main_body,
        grid=(_B // _BB,),
        in_specs=[
            pl.BlockSpec((_BB, _F, _D), lambda i: (i, 0, 0)),
            pl.BlockSpec((_BB, _F * _D), lambda i: (i, 0)),
            pl.BlockSpec((_BB, _P * _D), lambda i: (i, 0)),
            pl.BlockSpec((_F * _D, _OUT), lambda i: (0, 0)),
            pl.BlockSpec((_P * _D, _OUT), lambda i: (0, 0)),
            pl.BlockSpec((1, 1, _D), lambda i: (0, 0, 0)),
            pl.BlockSpec((_F, _F, _P), lambda i: (0, 0, 0)),
            pl.BlockSpec((_PC, _PC * _D), lambda i: (0, 0)),
            pl.BlockSpec((1, _OUT), lambda i: (0, 0)),
        ],
        out_specs=pl.BlockSpec((_BB, _OUT), lambda i: (i, 0)),
        out_shape=jax.ShapeDtypeStruct((_B, _OUT), jnp.float32),
    )(emb3, emb2, s2, W_out[:_F * _D], W_out[_F * _D:],
      w_attn.reshape(1, 1, _D), jnp.asarray(_SEL), jnp.asarray(_REP),
      b_out.reshape(1, _OUT))
    return out
